# Initial kernel scaffold; baseline (speedup 1.0000x reference)
import jax
import jax.numpy as jnp
from jax.experimental import pallas as pl


def kernel(positions, encoding_weight):
    raise NotImplementedError("write your pallas kernel here")


# speedup vs baseline: 3.0434x; 3.0434x over previous
import functools

import jax
import jax.numpy as jnp
from jax import lax
from jax.experimental import pallas as pl
from jax.experimental.pallas import tpu as pltpu
from jax.experimental.pallas import tpu_sc as plsc

D_MODEL = 64
NUM_CORES = 2
NUM_SUBCORES = 16
NUM_WORKERS = NUM_CORES * NUM_SUBCORES
GATHER = 128
K = 8
CHUNK = K * GATHER


@functools.partial(jax.jit, static_argnames=("n_rows",))
def _sc_gather(idx2d, table, *, n_rows):
    per_w = n_rows // NUM_WORKERS
    n_chunks = per_w // CHUNK
    mesh = plsc.VectorSubcoreMesh(
        core_axis_name="c", subcore_axis_name="s", num_cores=NUM_CORES
    )

    @functools.partial(
        pl.kernel,
        out_type=jax.ShapeDtypeStruct((n_rows, D_MODEL), jnp.float32),
        mesh=mesh,
        scratch_types=[
            pltpu.VMEM((K, GATHER), jnp.int32),
            pltpu.VMEM((CHUNK, D_MODEL), jnp.float32),
            pltpu.SemaphoreType.DMA,
        ],
        compiler_params=pltpu.CompilerParams(use_tc_tiling_on_sc=False),
    )
    def k(idx_hbm, table_hbm, out_hbm, idx_v, rows_v, sem):
        wid = lax.axis_index("s") * NUM_CORES + lax.axis_index("c")

        def chunk_body(c, _):
            row0 = pl.multiple_of(wid * per_w + c * CHUNK, CHUNK)
            irow0 = pl.multiple_of(row0 // GATHER, K)
            pltpu.sync_copy(idx_hbm.at[pl.ds(irow0, K)], idx_v)
            copies = [
                pltpu.async_copy(
                    table_hbm.at[idx_v.at[j]],
                    rows_v.at[pl.ds(j * GATHER, GATHER)],
                    sem,
                )
                for j in range(K)
            ]
            for cp in copies:
                cp.wait()
            pltpu.sync_copy(rows_v, out_hbm.at[pl.ds(row0, CHUNK)])
            return _

        lax.fori_loop(0, n_chunks, chunk_body, 0)

    return k(idx2d, table)


def kernel(positions, encoding_weight):
    bsz, hist = positions.shape
    _, d = encoding_weight.shape
    n = bsz * hist
    assert d == D_MODEL and n % (NUM_WORKERS * CHUNK) == 0
    idx2d = positions.reshape(n // GATHER, GATHER).astype(jnp.int32)
    out = _sc_gather(idx2d, encoding_weight, n_rows=n)
    return out.reshape(bsz, hist, d)


# speedup vs baseline: 3.0699x; 1.0087x over previous
import functools

import jax
import jax.numpy as jnp
from jax import lax
from jax.experimental import pallas as pl
from jax.experimental.pallas import tpu as pltpu
from jax.experimental.pallas import tpu_sc as plsc

D_MODEL = 64
NUM_CORES = 2
NUM_SUBCORES = 16
NUM_WORKERS = NUM_CORES * NUM_SUBCORES
GATHER = 128
GPC = 4
CHUNK = GPC * GATHER
CPB = 4
IDXROWS = CPB * GPC


@functools.partial(jax.jit, static_argnames=("n_rows",))
def _sc_gather(idx2d, table, *, n_rows):
    per_w = n_rows // NUM_WORKERS
    n_chunks = per_w // CHUNK
    n_blocks = n_chunks // CPB
    mesh = plsc.VectorSubcoreMesh(
        core_axis_name="c", subcore_axis_name="s", num_cores=NUM_CORES
    )

    @functools.partial(
        pl.kernel,
        out_type=jax.ShapeDtypeStruct((n_rows, D_MODEL), jnp.float32),
        mesh=mesh,
        scratch_types=[
            pltpu.VMEM((2, IDXROWS, GATHER), jnp.int32),
            pltpu.VMEM((2, CHUNK, D_MODEL), jnp.float32),
            pltpu.SemaphoreType.DMA,
            pltpu.SemaphoreType.DMA,
            pltpu.SemaphoreType.DMA,
        ],
        compiler_params=pltpu.CompilerParams(use_tc_tiling_on_sc=False),
    )
    def k(idx_hbm, table_hbm, out_hbm, idx_v, rows_v, sem_idx, sem_g, sem_out):
        wid = lax.axis_index("s") * NUM_CORES + lax.axis_index("c")
        row_base = pl.multiple_of(wid * per_w, CHUNK)
        irow_base = pl.multiple_of(row_base // GATHER, IDXROWS)

        def fire_chunk(idxbuf, local_chunk, rbuf):
            for j in range(GPC):
                pltpu.async_copy(
                    table_hbm.at[idx_v.at[idxbuf].at[local_chunk * GPC + j]],
                    rows_v.at[rbuf].at[pl.ds(j * GATHER, GATHER)],
                    sem_g,
                )

        def wait_bytes(sem, nbytes_ref_pair):
            src, dst = nbytes_ref_pair
            pltpu.make_async_copy(src, dst, sem).wait()

        pltpu.sync_copy(idx_hbm.at[pl.ds(irow_base, IDXROWS)], idx_v.at[0])
        fire_chunk(0, 0, 0)

        def blk_body(g, _):
            q = g % 2
            for h in range(CPB):
                b = h % 2
                c = g * CPB + h
                row0 = pl.multiple_of(row_base + c * CHUNK, CHUNK)
                out_slice = out_hbm.at[pl.ds(row0, CHUNK)]
                wait_bytes(sem_g, (out_slice, rows_v.at[b]))
                pltpu.async_copy(rows_v.at[b], out_slice, sem_out)
                if h == 0:
                    g1 = jnp.minimum(g + 1, n_blocks - 1)
                    ir = pl.multiple_of(irow_base + g1 * IDXROWS, IDXROWS)
                    pltpu.async_copy(
                        idx_hbm.at[pl.ds(ir, IDXROWS)], idx_v.at[1 - q], sem_idx
                    )

                @pl.when(c > 0)
                def _wait_prev_out():
                    wait_bytes(sem_out, (out_slice, rows_v.at[1 - b]))

                if h == CPB - 1:
                    wait_bytes(
                        sem_idx,
                        (idx_hbm.at[pl.ds(irow_base, IDXROWS)], idx_v.at[1 - q]),
                    )

                @pl.when(c + 1 < n_chunks)
                def _fire_next():
                    if h == CPB - 1:
                        fire_chunk(1 - q, 0, 1 - b)
                    else:
                        fire_chunk(q, h + 1, 1 - b)

            return _

        lax.fori_loop(0, n_blocks, blk_body, 0)
        last = pl.multiple_of(row_base + (n_chunks - 1) * CHUNK, CHUNK)
        pltpu.make_async_copy(
            rows_v.at[(n_chunks - 1) % 2],
            out_hbm.at[pl.ds(last, CHUNK)],
            sem_out,
        ).wait()

    return k(idx2d, table)


def kernel(positions, encoding_weight):
    bsz, hist = positions.shape
    _, d = encoding_weight.shape
    n = bsz * hist
    assert d == D_MODEL and n % (NUM_WORKERS * CHUNK * CPB) == 0
    idx2d = positions.reshape(n // GATHER, GATHER).astype(jnp.int32)
    out = _sc_gather(idx2d, encoding_weight, n_rows=n)
    return out.reshape(bsz, hist, d)
